# Initial kernel scaffold; baseline (speedup 1.0000x reference)
import jax
import jax.numpy as jnp
from jax.experimental import pallas as pl


def kernel(x, batch_ids, y, ln_g, ln_b, W, b):
    raise NotImplementedError("write your pallas kernel here")


# speedup vs baseline: 23.9568x; 23.9568x over previous
import functools

import jax
import jax.numpy as jnp
from jax import lax
from jax.experimental import pallas as pl
from jax.experimental.pallas import tpu as pltpu
from jax.experimental.pallas import tpu_sc as plsc

N = 50000
D = 256
B = 512
L = 16
NC = 2
NS = 16
CHUNK = 3136
N_PAD = NS * CHUNK
G_PER_W = B // (NC * NS)
RSQRT_MAGIC = 0x5F3759DF


def _rsqrt_newton(v):
    y = plsc.bitcast(RSQRT_MAGIC - lax.shift_right_logical(plsc.bitcast(v, jnp.int32), 1),
                     jnp.float32)
    half = v * 0.5
    for _ in range(3):
        y = y * (1.5 - half * y * y)
    return y


def _sc_body(x_hbm, ids_hbm, lng_hbm, lnb_hbm, w_hbm, bvec_hbm, out_hbm,
             ids_v, starts_v, merged_v, idx16_v, rows_v,
             g_v, bv_v, w_v, bias_v, out_v, shared, sem):
    c = lax.axis_index("c")
    s = lax.axis_index("s")
    w = c * NS + s

    zeros16 = jnp.zeros((L,), jnp.int32)
    iota = lax.iota(jnp.int32, L)
    for i in range(L):
        for j in range(8):
            starts_v[i, pl.ds(j * L, L)] = zeros16

    @pl.when(s == 0)
    def _():
        pltpu.sync_copy(starts_v, shared)

    plsc.subcore_barrier()

    base = s * CHUNK
    pltpu.sync_copy(ids_hbm.at[pl.ds(base, CHUNK)], ids_v.at[pl.ds(L, CHUNK)])

    @pl.when(s == 0)
    def _():
        ids_v[pl.ds(0, L)] = jnp.full((L,), -1, jnp.int32)

    @pl.when(s > 0)
    def _():
        pltpu.sync_copy(ids_hbm.at[pl.ds(base - L, L)], ids_v.at[pl.ds(0, L)])

    def scan_step(k, _):
        off = k * L
        curr = ids_v[pl.ds(off + L, L)]
        prev = ids_v[pl.ds(off + L - 1, L)]
        pos = base + off + iota
        mask = (curr != prev) & (pos < N)
        g = jnp.clip(curr, 0, B - 1)
        plsc.addupdate_scatter(
            starts_v,
            [lax.shift_right_logical(g, 5), lax.bitwise_and(g, 31)],
            pos + 1, mask=mask)
        return 0

    lax.fori_loop(0, CHUNK // L, scan_step, 0)

    pltpu.sync_copy(starts_v, shared.at[iota], add=True)
    plsc.subcore_barrier()

    pltpu.sync_copy(shared.at[w >> 1, pl.ds((w & 1) * L, L)], merged_v)
    mv = merged_v[...]
    valid = mv > 0
    idx16_v[...] = jnp.maximum(mv - 1, 0)

    pltpu.async_copy(x_hbm.at[idx16_v], rows_v, sem).wait()

    pltpu.sync_copy(lng_hbm, g_v)
    pltpu.sync_copy(lnb_hbm, bv_v)
    pltpu.sync_copy(w_hbm, w_v)
    pltpu.sync_copy(bvec_hbm, bias_v)

    inv_d = 1.0 / D
    preds = jnp.zeros((L,), jnp.float32)
    for r in range(G_PER_W):
        def mom_step(d, carry):
            acc, acc2 = carry
            xv = rows_v[r, pl.ds(d * L, L)]
            return acc + xv, acc2 + xv * xv

        z = jnp.zeros((L,), jnp.float32)
        acc, acc2 = lax.fori_loop(0, D // L, mom_step, (z, z))
        mu = jnp.sum(acc) * inv_d
        var = jnp.sum(acc2) * inv_d - mu * mu
        mu_v = jnp.full((L,), mu, jnp.float32)
        rstd_v = _rsqrt_newton(jnp.full((L,), var + 1e-5, jnp.float32))

        def dot_step(d, dot):
            xv = rows_v[r, pl.ds(d * L, L)]
            xn = (xv - mu_v) * rstd_v * g_v[pl.ds(d * L, L)] + bv_v[pl.ds(d * L, L)]
            return dot + xn * w_v[pl.ds(d * L, L)]

        dot = lax.fori_loop(0, D // L, dot_step, z)
        preds = jnp.where(iota == r, jnp.sum(dot), preds)

    res = jnp.where(valid, preds, 0.0) + bias_v[...]
    out_v[...] = res
    pltpu.sync_copy(out_v, out_hbm.at[pl.ds(w * G_PER_W, G_PER_W)])


@jax.jit
def _run(x, ids_pad, ln_g, ln_b, w_flat, b_vec):
    mesh = plsc.VectorSubcoreMesh(core_axis_name="c", subcore_axis_name="s")
    f = pl.kernel(
        _sc_body, mesh=mesh,
        out_type=jax.ShapeDtypeStruct((B,), jnp.float32),
        scratch_types=[
            pltpu.VMEM((CHUNK + L,), jnp.int32),
            pltpu.VMEM((L, 128), jnp.int32),
            pltpu.VMEM((L,), jnp.int32),
            pltpu.VMEM((L,), jnp.int32),
            pltpu.VMEM((G_PER_W, D), jnp.float32),
            pltpu.VMEM((D,), jnp.float32),
            pltpu.VMEM((D,), jnp.float32),
            pltpu.VMEM((D,), jnp.float32),
            pltpu.VMEM((L,), jnp.float32),
            pltpu.VMEM((L,), jnp.float32),
            pltpu.VMEM_SHARED((L, 128), jnp.int32),
            pltpu.SemaphoreType.DMA,
        ],
        compiler_params=pltpu.CompilerParams(needs_layout_passes=False),
    )
    return f(x, ids_pad, ln_g, ln_b, w_flat, b_vec)


def kernel(x, batch_ids, y, ln_g, ln_b, W, b):
    ids32 = batch_ids.astype(jnp.int32)
    ids_pad = jnp.concatenate(
        [ids32, jnp.full((N_PAD - N,), -1, jnp.int32)])
    pred = _run(x, ids_pad, ln_g, ln_b, W.reshape(D), jnp.broadcast_to(b, (L,)))
    return (pred.reshape(B, 1), y)


# speedup vs baseline: 30.5511x; 1.2753x over previous
import jax
import jax.numpy as jnp
from jax import lax
from jax.experimental import pallas as pl
from jax.experimental.pallas import tpu as pltpu
from jax.experimental.pallas import tpu_sc as plsc

N = 50000
D = 256
B = 512
L = 16
NC = 2
NS = 16
CHUNK = 3136
N_PAD = NS * CHUNK
G_PER_W = B // (NC * NS)
RSQRT_MAGIC = 0x5F3759DF


def _rsqrt_newton(v):
    y = plsc.bitcast(RSQRT_MAGIC - lax.shift_right_logical(plsc.bitcast(v, jnp.int32), 1),
                     jnp.float32)
    half = v * 0.5
    for _ in range(3):
        y = y * (1.5 - half * y * y)
    return y


def _sc_body(x_hbm, ids_hbm, lng_hbm, lnb_hbm, w_hbm, bvec_hbm, out_hbm,
             ids_v, starts_v, merged_v, idx16_v, rows_v,
             g_v, bv_v, w_v, bias_v, out_v, shared, sem, sem2):
    c = lax.axis_index("c")
    s = lax.axis_index("s")
    w = c * NS + s
    iota = lax.iota(jnp.int32, L)
    base = s * CHUNK

    cd_ids = pltpu.async_copy(
        ids_hbm.at[pl.ds(base, CHUNK)], ids_v.at[pl.ds(L, CHUNK)], sem)
    cd_g = pltpu.async_copy(lng_hbm, g_v, sem2)
    cd_b = pltpu.async_copy(lnb_hbm, bv_v, sem2)
    cd_w = pltpu.async_copy(w_hbm, w_v, sem2)
    cd_bias = pltpu.async_copy(bvec_hbm, bias_v, sem2)

    @pl.when(s == 0)
    def _():
        ids_v[pl.ds(0, L)] = jnp.full((L,), -1, jnp.int32)

    @pl.when(s > 0)
    def _():
        pltpu.sync_copy(ids_hbm.at[pl.ds(base - L, L)], ids_v.at[pl.ds(0, L)])

    zeros16 = jnp.zeros((L,), jnp.int32)
    for i in range(L):
        for j in range(8):
            starts_v[i, pl.ds(j * L, L)] = zeros16

    @pl.when(s == 0)
    def _():
        pltpu.sync_copy(starts_v, shared)

    plsc.subcore_barrier()
    cd_ids.wait()

    @plsc.parallel_loop(0, CHUNK // L, unroll=4)
    def _scan(k):
        off = k * L
        curr = ids_v[pl.ds(off + L, L)]
        prev = ids_v[pl.ds(off + L - 1, L)]
        pos = base + off + iota
        mask = (curr != prev) & (pos < N)
        g = jnp.clip(curr, 0, B - 1)
        plsc.addupdate_scatter(
            starts_v,
            [lax.shift_right_logical(g, 5), lax.bitwise_and(g, 31)],
            pos + 1, mask=mask)

    pltpu.sync_copy(starts_v, shared.at[iota], add=True)
    plsc.subcore_barrier()

    pltpu.sync_copy(shared.at[w >> 1, pl.ds((w & 1) * L, L)], merged_v)
    mv = merged_v[...]
    valid = mv > 0
    idx16_v[...] = jnp.maximum(mv - 1, 0)

    cd_rows = pltpu.async_copy(x_hbm.at[idx16_v], rows_v, sem)

    cd_g.wait(); cd_b.wait(); cd_w.wait(); cd_bias.wait()

    gw = []
    gsum_acc = jnp.zeros((L,), jnp.float32)
    bsum_acc = jnp.zeros((L,), jnp.float32)
    for d in range(D // L):
        wv = w_v[pl.ds(d * L, L)]
        gwd = g_v[pl.ds(d * L, L)] * wv
        gw.append(gwd)
        gsum_acc = gsum_acc + gwd
        bsum_acc = bsum_acc + bv_v[pl.ds(d * L, L)] * wv
    gsum = jnp.sum(gsum_acc)
    bsum = jnp.sum(bsum_acc)

    cd_rows.wait()

    inv_d = 1.0 / D
    preds = jnp.zeros((L,), jnp.float32)
    for r in range(G_PER_W):
        z = jnp.zeros((L,), jnp.float32)
        acc, acc2, acca = z, z, z
        for d in range(D // L):
            xv = rows_v[r, pl.ds(d * L, L)]
            acc = acc + xv
            acc2 = acc2 + xv * xv
            acca = acca + xv * gw[d]
        mu = jnp.sum(acc) * inv_d
        var = jnp.sum(acc2) * inv_d - mu * mu
        a = jnp.sum(acca)
        rstd_v = _rsqrt_newton(jnp.full((L,), var + 1e-5, jnp.float32))
        pred_v = (a - mu * gsum) * rstd_v + bsum
        preds = jnp.where(iota == r, pred_v, preds)

    res = jnp.where(valid, preds, 0.0) + bias_v[...]
    out_v[...] = res
    pltpu.sync_copy(out_v, out_hbm.at[pl.ds(w * G_PER_W, G_PER_W)])


@jax.jit
def _run(x, ids_pad, ln_g, ln_b, w_flat, b_vec):
    mesh = plsc.VectorSubcoreMesh(core_axis_name="c", subcore_axis_name="s")
    f = pl.kernel(
        _sc_body, mesh=mesh,
        out_type=jax.ShapeDtypeStruct((B,), jnp.float32),
        scratch_types=[
            pltpu.VMEM((CHUNK + L,), jnp.int32),
            pltpu.VMEM((L, 128), jnp.int32),
            pltpu.VMEM((L,), jnp.int32),
            pltpu.VMEM((L,), jnp.int32),
            pltpu.VMEM((G_PER_W, D), jnp.float32),
            pltpu.VMEM((D,), jnp.float32),
            pltpu.VMEM((D,), jnp.float32),
            pltpu.VMEM((D,), jnp.float32),
            pltpu.VMEM((L,), jnp.float32),
            pltpu.VMEM((L,), jnp.float32),
            pltpu.VMEM_SHARED((L, 128), jnp.int32),
            pltpu.SemaphoreType.DMA,
            pltpu.SemaphoreType.DMA,
        ],
        compiler_params=pltpu.CompilerParams(needs_layout_passes=False),
    )
    return f(x, ids_pad, ln_g, ln_b, w_flat, b_vec)


def kernel(x, batch_ids, y, ln_g, ln_b, W, b):
    ids32 = batch_ids.astype(jnp.int32)
    ids_pad = jnp.concatenate(
        [ids32, jnp.full((N_PAD - N,), -1, jnp.int32)])
    pred = _run(x, ids_pad, ln_g, ln_b, W.reshape(D), jnp.broadcast_to(b, (L,)))
    return (pred.reshape(B, 1), y)


# speedup vs baseline: 31.6781x; 1.0369x over previous
import jax
import jax.numpy as jnp
from jax import lax
from jax.experimental import pallas as pl
from jax.experimental.pallas import tpu as pltpu
from jax.experimental.pallas import tpu_sc as plsc

N = 50000
D = 256
B = 512
L = 16
NC = 2
NS = 16
CHUNK = 3136
TAIL = N - (NS - 1) * CHUNK
G_PER_W = B // (NC * NS)
RSQRT_MAGIC = 0x5F3759DF


def _rsqrt_newton(v):
    y = plsc.bitcast(RSQRT_MAGIC - lax.shift_right_logical(plsc.bitcast(v, jnp.int32), 1),
                     jnp.float32)
    half = v * 0.5
    for _ in range(3):
        y = y * (1.5 - half * y * y)
    return y


def _sc_body(x_hbm, ids_hbm, lng_hbm, lnb_hbm, w_hbm, bvec_hbm, out_hbm,
             ids_v, starts_v, merged_v, idx16_v, rows_v,
             g_v, bv_v, w_v, bias_v, out_v, shared, sem, sem2):
    c = lax.axis_index("c")
    s = lax.axis_index("s")
    w = c * NS + s
    iota = lax.iota(jnp.int32, L)
    base = s * CHUNK

    @pl.when(s < NS - 1)
    def _():
        pltpu.async_copy(
            ids_hbm.at[pl.ds(base, CHUNK)], ids_v.at[pl.ds(L, CHUNK)], sem)

    @pl.when(s == NS - 1)
    def _():
        pltpu.async_copy(
            ids_hbm.at[pl.ds(base, TAIL)], ids_v.at[pl.ds(L, TAIL)], sem)

    cd_g = pltpu.async_copy(lng_hbm, g_v, sem2)
    cd_b = pltpu.async_copy(lnb_hbm, bv_v, sem2)
    cd_w = pltpu.async_copy(w_hbm, w_v, sem2)
    cd_bias = pltpu.async_copy(bvec_hbm, bias_v, sem2)

    @pl.when(s == 0)
    def _():
        ids_v[pl.ds(0, L)] = jnp.full((L,), -1, jnp.int32)

    @pl.when(s > 0)
    def _():
        pltpu.sync_copy(ids_hbm.at[pl.ds(base - L, L)], ids_v.at[pl.ds(0, L)])

    zeros16 = jnp.zeros((L,), jnp.int32)

    @plsc.parallel_loop(0, L, unroll=2)
    def _zero(i):
        for j in range(8):
            starts_v[i, pl.ds(j * L, L)] = zeros16

    @pl.when(s == 0)
    def _():
        pltpu.sync_copy(starts_v, shared)

    plsc.subcore_barrier()

    @pl.when(s < NS - 1)
    def _():
        pltpu.make_async_copy(
            ids_hbm.at[pl.ds(base, CHUNK)], ids_v.at[pl.ds(L, CHUNK)], sem).wait()

    @pl.when(s == NS - 1)
    def _():
        pltpu.make_async_copy(
            ids_hbm.at[pl.ds(base, TAIL)], ids_v.at[pl.ds(L, TAIL)], sem).wait()

    @plsc.parallel_loop(0, CHUNK // L, unroll=4)
    def _scan(k):
        off = k * L
        curr = ids_v[pl.ds(off + L, L)]
        prev = ids_v[pl.ds(off + L - 1, L)]
        pos = base + off + iota
        mask = (curr != prev) & (pos < N)
        g = jnp.clip(curr, 0, B - 1)
        plsc.addupdate_scatter(
            starts_v,
            [lax.shift_right_logical(g, 5), lax.bitwise_and(g, 31)],
            pos + 1, mask=mask)

    pltpu.sync_copy(starts_v, shared.at[iota], add=True)
    plsc.subcore_barrier()

    pltpu.sync_copy(shared.at[w >> 1, pl.ds((w & 1) * L, L)], merged_v)
    mv = merged_v[...]
    valid = mv > 0
    idx16_v[...] = jnp.maximum(mv - 1, 0)

    cd_rows = pltpu.async_copy(x_hbm.at[idx16_v], rows_v, sem)

    cd_g.wait(); cd_b.wait(); cd_w.wait(); cd_bias.wait()

    gw = []
    gsum_acc = jnp.zeros((L,), jnp.float32)
    bsum_acc = jnp.zeros((L,), jnp.float32)
    for d in range(D // L):
        wv = w_v[pl.ds(d * L, L)]
        gwd = g_v[pl.ds(d * L, L)] * wv
        gw.append(gwd)
        gsum_acc = gsum_acc + gwd
        bsum_acc = bsum_acc + bv_v[pl.ds(d * L, L)] * wv
    gsum = jnp.sum(gsum_acc)
    bsum = jnp.sum(bsum_acc)

    cd_rows.wait()

    inv_d = 1.0 / D

    @plsc.parallel_loop(0, G_PER_W, unroll=2, carry=jnp.zeros((L,), jnp.float32))
    def preds(r, acc_preds):
        z = jnp.zeros((L,), jnp.float32)
        acc, acc2, acca = z, z, z
        for d in range(D // L):
            xv = rows_v[r, pl.ds(d * L, L)]
            acc = acc + xv
            acc2 = acc2 + xv * xv
            acca = acca + xv * gw[d]
        mu = jnp.sum(acc) * inv_d
        var = jnp.sum(acc2) * inv_d - mu * mu
        a = jnp.sum(acca)
        rstd_v = _rsqrt_newton(jnp.full((L,), var + 1e-5, jnp.float32))
        pred_v = (a - mu * gsum) * rstd_v + bsum
        return jnp.where(iota == r, pred_v, acc_preds)

    res = jnp.where(valid, preds, 0.0) + bias_v[...]
    out_v[...] = res
    pltpu.sync_copy(out_v, out_hbm.at[pl.ds(w * G_PER_W, G_PER_W)])


@jax.jit
def _run(x, ids_pad, ln_g, ln_b, w_flat, b_vec):
    mesh = plsc.VectorSubcoreMesh(core_axis_name="c", subcore_axis_name="s")
    f = pl.kernel(
        _sc_body, mesh=mesh,
        out_type=jax.ShapeDtypeStruct((B,), jnp.float32),
        scratch_types=[
            pltpu.VMEM((CHUNK + L,), jnp.int32),
            pltpu.VMEM((L, 128), jnp.int32),
            pltpu.VMEM((L,), jnp.int32),
            pltpu.VMEM((L,), jnp.int32),
            pltpu.VMEM((G_PER_W, D), jnp.float32),
            pltpu.VMEM((D,), jnp.float32),
            pltpu.VMEM((D,), jnp.float32),
            pltpu.VMEM((D,), jnp.float32),
            pltpu.VMEM((L,), jnp.float32),
            pltpu.VMEM((L,), jnp.float32),
            pltpu.VMEM_SHARED((L, 128), jnp.int32),
            pltpu.SemaphoreType.DMA,
            pltpu.SemaphoreType.DMA,
        ],
        compiler_params=pltpu.CompilerParams(needs_layout_passes=False),
    )
    return f(x, ids_pad, ln_g, ln_b, w_flat, b_vec)


def kernel(x, batch_ids, y, ln_g, ln_b, W, b):
    ids32 = batch_ids.astype(jnp.int32)
    pred = _run(x, ids32, ln_g, ln_b, W.reshape(D), jnp.broadcast_to(b, (L,)))
    return (pred.reshape(B, 1), y)
